# Initial kernel scaffold; baseline (speedup 1.0000x reference)
import jax
import jax.numpy as jnp
from jax.experimental import pallas as pl


def kernel(x, edge_index, W1, b1, W2, b2, W3, b3, Wc, bc):
    raise NotImplementedError("write your pallas kernel here")


# speedup vs baseline: 55.8006x; 55.8006x over previous
import functools

import jax
import jax.numpy as jnp
from jax import lax
from jax.experimental import pallas as pl
from jax.experimental.pallas import tpu as pltpu
from jax.experimental.pallas import tpu_sc as plsc

N = 10000
D_IN = 128
E = 320000
NPAD = 10240
W = 128
EPAD = 331776
NW = EPAD // (32 * W)
EC = NW * W
NS = NPAD // 16
NG = NPAD // 32

_MESH = plsc.VectorSubcoreMesh(
    core_axis_name="c", subcore_axis_name="s", num_cores=2, num_subcores=16)
_SC_PARAMS = pltpu.CompilerParams(
    use_tc_tiling_on_sc=False, needs_layout_passes=False)

_f32 = jnp.float32
_i32 = jnp.int32


def _rsqrt16(d):
    i = lax.bitcast_convert_type(d, _i32)
    i = jnp.int32(0x5F3759DF) - lax.shift_right_arithmetic(i, 1)
    y = lax.bitcast_convert_type(i, _f32)
    h = d * 0.5
    for _ in range(3):
        y = y * (1.5 - h * y * y)
    return y


def _fill(ref, n, val):
    v = jnp.full((16,), val, _f32)
    for k in range(n // 16):
        ref[pl.ds(k * 16, 16)] = v


def _gather_scatter(u_s, acc_s, srcflat, gbuf, dstbuf, sem, nf):
    gd = [pltpu.async_copy(u_s[f].at[srcflat], gbuf[f], sem) for f in range(nf)]
    for d in gd:
        d.wait()
    for f in range(nf):
        for g in range(0, NW, 16):
            ds_ = [
                pltpu.async_copy(
                    gbuf[f].at[pl.ds(w * W, W)],
                    acc_s[f].at[dstbuf.at[w]],
                    sem, add=True)
                for w in range(g, min(g + 16, NW))
            ]
            for d in ds_:
                d.wait()


_MMBLK = 1024


def _mm1_body(x_ref, w1t_ref, o_ref):
    xb = x_ref[...]
    for f in range(4):
        wrow = w1t_ref[f, :]
        o_ref[pl.ds(f, 1), :] = jnp.sum(xb * wrow[None, :], axis=1)[None, :]


_mm1 = pl.pallas_call(
    _mm1_body,
    grid=(NPAD // _MMBLK,),
    in_specs=[
        pl.BlockSpec((_MMBLK, D_IN), lambda i: (i, 0)),
        pl.BlockSpec((4, D_IN), lambda i: (0, 0)),
    ],
    out_specs=pl.BlockSpec((4, _MMBLK), lambda i: (0, i)),
    out_shape=jax.ShapeDtypeStruct((4, NPAD), _f32),
)


@functools.partial(
    pl.kernel,
    mesh=_MESH,
    compiler_params=_SC_PARAMS,
    out_type=jax.ShapeDtypeStruct((2, NPAD), _f32),
    scratch_types=[
        pltpu.VMEM((NW, W), _i32),
        pltpu.VMEM((W,), _f32),
        pltpu.VMEM((NS,), _f32),
        pltpu.VMEM_SHARED((NPAD,), _f32),
        pltpu.SemaphoreType.DMA,
    ],
)
def _p1(dst2d, degp, dstbuf, ones, zb, degs, sem):
    cid = lax.axis_index("c")
    sid = lax.axis_index("s")
    wid = cid * 16 + sid
    s0 = pl.ds(sid * NS, NS)
    _fill(ones, W, 1.0)
    _fill(zb, NS, 0.0)
    pltpu.sync_copy(zb, degs.at[s0])
    pltpu.sync_copy(dst2d.at[pl.ds(wid * NW, NW)], dstbuf)
    plsc.subcore_barrier()
    for g in range(0, NW, 16):
        ds_ = [
            pltpu.async_copy(ones, degs.at[dstbuf.at[w]], sem, add=True)
            for w in range(g, min(g + 16, NW))
        ]
        for d in ds_:
            d.wait()
    plsc.subcore_barrier()
    pltpu.sync_copy(degs.at[s0], degp.at[cid, s0])


@functools.partial(
    pl.kernel,
    mesh=_MESH,
    compiler_params=_SC_PARAMS,
    out_type=[
        jax.ShapeDtypeStruct((NPAD,), _f32),
        jax.ShapeDtypeStruct((2, 4, NPAD), _f32),
    ],
    scratch_types=[
        pltpu.VMEM((EC,), _i32),
        pltpu.VMEM((NW, W), _i32),
        [pltpu.VMEM((EC,), _f32) for _ in range(4)],
        pltpu.VMEM((NS,), _f32),
        pltpu.VMEM((NS,), _f32),
        pltpu.VMEM((NS,), _f32),
        [pltpu.VMEM_SHARED((NPAD,), _f32) for _ in range(4)],
        [pltpu.VMEM_SHARED((NPAD,), _f32) for _ in range(4)],
        pltpu.SemaphoreType.DMA,
    ],
)
def _l1(src1d, dst2d, degp, xw1t, dinv_o, accp, srcflat, dstbuf, gbuf,
        nb0, nb1, zb, u_s, acc_s, sem):
    cid = lax.axis_index("c")
    sid = lax.axis_index("s")
    wid = cid * 16 + sid
    s0 = pl.ds(sid * NS, NS)
    pltpu.sync_copy(degp.at[0, s0], nb0)
    pltpu.sync_copy(degp.at[1, s0], nb1)
    for k in range(NS // 16):
        ds_ = pl.ds(k * 16, 16)
        nb0[ds_] = _rsqrt16(nb0[ds_] + nb1[ds_] + 1.0)

    @pl.when(cid == 0)
    def _():
        pltpu.sync_copy(nb0, dinv_o.at[s0])

    _fill(zb, NS, 0.0)
    for f in range(4):
        pltpu.sync_copy(xw1t.at[f, s0], nb1)
        for k in range(NS // 16):
            ds_ = pl.ds(k * 16, 16)
            nb1[ds_] = nb1[ds_] * nb0[ds_]
        pltpu.sync_copy(nb1, u_s[f].at[s0])
        pltpu.sync_copy(zb, acc_s[f].at[s0])
    pltpu.sync_copy(src1d.at[pl.ds(wid * EC, EC)], srcflat)
    pltpu.sync_copy(dst2d.at[pl.ds(wid * NW, NW)], dstbuf)
    plsc.subcore_barrier()
    _gather_scatter(u_s, acc_s, srcflat, gbuf, dstbuf, sem, 4)
    plsc.subcore_barrier()
    for f in range(4):
        pltpu.sync_copy(acc_s[f].at[s0], accp.at[cid, f, s0])


def _mid_layer(din, dout):
    @functools.partial(
        pl.kernel,
        mesh=_MESH,
        compiler_params=_SC_PARAMS,
        out_type=jax.ShapeDtypeStruct((2, dout, NPAD), _f32),
        scratch_types=[
            pltpu.VMEM((EC,), _i32),
            pltpu.VMEM((NW, W), _i32),
            [pltpu.VMEM((EC,), _f32) for _ in range(dout)],
            pltpu.VMEM((din, NS), _f32),
            pltpu.VMEM((NS,), _f32),
            pltpu.VMEM((NS,), _f32),
            pltpu.VMEM((NS,), _f32),
            pltpu.VMEM((din, dout, 16), _f32),
            pltpu.VMEM((din, 16), _f32),
            [pltpu.VMEM_SHARED((NPAD,), _f32) for _ in range(dout)],
            [pltpu.VMEM_SHARED((NPAD,), _f32) for _ in range(dout)],
            pltpu.SemaphoreType.DMA,
        ],
    )
    def _lay(src1d, dst2d, accp_in, dinv, wb, bb, accp_out, srcflat, dstbuf,
             gbuf, ybuf, t0, t1, dv, wbuf, bbuf, u_s, acc_s, sem):
        cid = lax.axis_index("c")
        sid = lax.axis_index("s")
        wid = cid * 16 + sid
        s0 = pl.ds(sid * NS, NS)
        pltpu.sync_copy(wb, wbuf)
        pltpu.sync_copy(bb, bbuf)
        pltpu.sync_copy(dinv.at[s0], dv)
        for f in range(din):
            pltpu.sync_copy(accp_in.at[0, f, s0], t0)
            pltpu.sync_copy(accp_in.at[1, f, s0], t1)
            for k in range(NS // 16):
                ds_ = pl.ds(k * 16, 16)
                ybuf[f, ds_] = jnp.maximum(
                    (t0[ds_] + t1[ds_]) * dv[ds_] + bbuf[f], 0.0)
        _fill(t1, NS, 0.0)
        for fp in range(dout):
            for k in range(NS // 16):
                ds_ = pl.ds(k * 16, 16)
                a = ybuf[0, ds_] * wbuf[0, fp]
                for f in range(1, din):
                    a = a + ybuf[f, ds_] * wbuf[f, fp]
                t0[ds_] = a * dv[ds_]
            pltpu.sync_copy(t0, u_s[fp].at[s0])
            pltpu.sync_copy(t1, acc_s[fp].at[s0])
        pltpu.sync_copy(src1d.at[pl.ds(wid * EC, EC)], srcflat)
        pltpu.sync_copy(dst2d.at[pl.ds(wid * NW, NW)], dstbuf)
        plsc.subcore_barrier()
        _gather_scatter(u_s, acc_s, srcflat, gbuf, dstbuf, sem, dout)
        plsc.subcore_barrier()
        for fp in range(dout):
            pltpu.sync_copy(acc_s[fp].at[s0], accp_out.at[cid, fp, s0])

    return _lay


_l2 = _mid_layer(4, 4)
_l3 = _mid_layer(4, 2)


@functools.partial(
    pl.kernel,
    mesh=_MESH,
    compiler_params=_SC_PARAMS,
    out_type=[
        jax.ShapeDtypeStruct((NPAD * 4,), _f32),
        jax.ShapeDtypeStruct((NPAD * 2,), _f32),
    ],
    scratch_types=[
        pltpu.VMEM((2, NG), _f32),
        pltpu.VMEM((NG,), _f32),
        pltpu.VMEM((NG,), _f32),
        pltpu.VMEM((NG,), _f32),
        pltpu.VMEM((2, 16), _f32),
        pltpu.VMEM((2, 4, 16), _f32),
        pltpu.VMEM((4, 16), _f32),
        pltpu.VMEM((NG * 4,), _f32),
        pltpu.VMEM((NG * 2,), _f32),
    ],
)
def _fin(accp3, dinv, b3b, wcb, bcb, out_o, y3_o,
         ybuf, t0, t1, dv, b3v, wcv, bcv, oil, yil):
    cid = lax.axis_index("c")
    sid = lax.axis_index("s")
    wid = cid * 16 + sid
    sg = pl.ds(wid * NG, NG)
    pltpu.sync_copy(b3b, b3v)
    pltpu.sync_copy(wcb, wcv)
    pltpu.sync_copy(bcb, bcv)
    pltpu.sync_copy(dinv.at[sg], dv)
    for f in range(2):
        pltpu.sync_copy(accp3.at[0, f, sg], t0)
        pltpu.sync_copy(accp3.at[1, f, sg], t1)
        for k in range(NG // 16):
            ds_ = pl.ds(k * 16, 16)
            ybuf[f, ds_] = jnp.maximum(
                (t0[ds_] + t1[ds_]) * dv[ds_] + b3v[f], 0.0)
    lane = lax.iota(_i32, 16)
    for k in range(NG // 16):
        ds_ = pl.ds(k * 16, 16)
        y0 = ybuf[0, ds_]
        y1 = ybuf[1, ds_]
        plsc.store_scatter(yil, [lane * 2 + (k * 32 + 0)], y0)
        plsc.store_scatter(yil, [lane * 2 + (k * 32 + 1)], y1)
        for c in range(4):
            oc = y0 * wcv[0, c] + y1 * wcv[1, c] + bcv[c]
            plsc.store_scatter(oil, [lane * 4 + (k * 64 + c)], oc)
    pltpu.sync_copy(oil, out_o.at[pl.ds(wid * NG * 4, NG * 4)])
    pltpu.sync_copy(yil, y3_o.at[pl.ds(wid * NG * 2, NG * 2)])


def kernel(x, edge_index, W1, b1, W2, b2, W3, b3, Wc, bc):
    xpad = jnp.zeros((NPAD, D_IN), _f32).at[:N].set(x)
    loops = jnp.arange(NPAD, dtype=_i32)
    npad_e = EPAD - E - NPAD
    padv = N + (jnp.arange(npad_e, dtype=_i32) % (NPAD - N))
    src = jnp.concatenate([edge_index[0], loops, padv])
    dst = jnp.concatenate([edge_index[1], loops, padv])
    dst2d = dst.reshape(EPAD // W, W)

    w2b = jnp.broadcast_to(W2[:, :, None], (4, 4, 16)).astype(_f32)
    w3b = jnp.broadcast_to(W3[:, :, None], (4, 2, 16)).astype(_f32)
    wcb = jnp.broadcast_to(Wc[:, :, None], (2, 4, 16)).astype(_f32)
    b1b = jnp.broadcast_to(b1[:, None], (4, 16)).astype(_f32)
    b2b = jnp.broadcast_to(b2[:, None], (4, 16)).astype(_f32)
    b3b = jnp.broadcast_to(b3[:, None], (2, 16)).astype(_f32)
    bcb = jnp.broadcast_to(bc[:, None], (4, 16)).astype(_f32)

    degp = _p1(dst2d)
    xw1t = _mm1(xpad, W1.T.astype(_f32))
    dinv, accp1 = _l1(src, dst2d, degp, xw1t)
    accp2 = _l2(src, dst2d, accp1, dinv, w2b, b1b)
    accp3 = _l3(src, dst2d, accp2, dinv, w3b, b2b)
    outf, y3f = _fin(accp3, dinv, b3b, wcb, bcb)
    out = outf.reshape(NPAD, 4)[:N]
    y3 = y3f.reshape(NPAD, 2)[:N]
    return (out, y3)


# speedup vs baseline: 56.7078x; 1.0163x over previous
import functools

import jax
import jax.numpy as jnp
from jax import lax
from jax.experimental import pallas as pl
from jax.experimental.pallas import tpu as pltpu
from jax.experimental.pallas import tpu_sc as plsc

N = 10000
D_IN = 128
E = 320000
NPAD = 10240
W = 128
EPAD = 331776
NW = EPAD // (32 * W)
EC = NW * W
NS = NPAD // 16
NG = NPAD // 32

_MESH = plsc.VectorSubcoreMesh(
    core_axis_name="c", subcore_axis_name="s", num_cores=2, num_subcores=16)
_SC_PARAMS = pltpu.CompilerParams(
    use_tc_tiling_on_sc=False, needs_layout_passes=False)

_f32 = jnp.float32
_i32 = jnp.int32


def _rsqrt16(d):
    i = lax.bitcast_convert_type(d, _i32)
    i = jnp.int32(0x5F3759DF) - lax.shift_right_arithmetic(i, 1)
    y = lax.bitcast_convert_type(i, _f32)
    h = d * 0.5
    for _ in range(3):
        y = y * (1.5 - h * y * y)
    return y


def _fill(ref, n, val):
    v = jnp.full((16,), val, _f32)
    for k in range(n // 16):
        ref[pl.ds(k * 16, 16)] = v


def _gather_scatter(u_s, acc_s, srcflat, gbuf, dstbuf, sem, nf):
    gd = [pltpu.async_copy(u_s[f].at[srcflat], gbuf[f], sem) for f in range(nf)]
    for f in range(nf):
        gd[f].wait()
        sd = [
            pltpu.async_copy(
                gbuf[f].at[pl.ds(w * W, W)],
                acc_s[f].at[dstbuf.at[w]],
                sem, add=True)
            for w in range(NW)
        ]
        for d in sd:
            d.wait()


_MMBLK = 1024


def _mm1_body(x_ref, w1t_ref, o_ref):
    xb = x_ref[...]
    for f in range(4):
        wrow = w1t_ref[f, :]
        o_ref[pl.ds(f, 1), :] = jnp.sum(xb * wrow[None, :], axis=1)[None, :]


_mm1 = pl.pallas_call(
    _mm1_body,
    grid=(NPAD // _MMBLK,),
    in_specs=[
        pl.BlockSpec((_MMBLK, D_IN), lambda i: (i, 0)),
        pl.BlockSpec((4, D_IN), lambda i: (0, 0)),
    ],
    out_specs=pl.BlockSpec((4, _MMBLK), lambda i: (0, i)),
    out_shape=jax.ShapeDtypeStruct((4, NPAD), _f32),
)


@functools.partial(
    pl.kernel,
    mesh=_MESH,
    compiler_params=_SC_PARAMS,
    out_type=jax.ShapeDtypeStruct((2, NPAD), _f32),
    scratch_types=[
        pltpu.VMEM((NW, W), _i32),
        pltpu.VMEM((W,), _f32),
        pltpu.VMEM((NS,), _f32),
        pltpu.VMEM_SHARED((NPAD,), _f32),
        pltpu.SemaphoreType.DMA,
    ],
)
def _p1(dst2d, degp, dstbuf, ones, zb, degs, sem):
    cid = lax.axis_index("c")
    sid = lax.axis_index("s")
    wid = cid * 16 + sid
    s0 = pl.ds(sid * NS, NS)
    _fill(ones, W, 1.0)
    _fill(zb, NS, 0.0)
    pltpu.sync_copy(zb, degs.at[s0])
    pltpu.sync_copy(dst2d.at[pl.ds(wid * NW, NW)], dstbuf)
    plsc.subcore_barrier()
    sd = [
        pltpu.async_copy(ones, degs.at[dstbuf.at[w]], sem, add=True)
        for w in range(NW)
    ]
    for d in sd:
        d.wait()
    plsc.subcore_barrier()
    pltpu.sync_copy(degs.at[s0], degp.at[cid, s0])


@functools.partial(
    pl.kernel,
    mesh=_MESH,
    compiler_params=_SC_PARAMS,
    out_type=[
        jax.ShapeDtypeStruct((NPAD,), _f32),
        jax.ShapeDtypeStruct((2, 4, NPAD), _f32),
    ],
    scratch_types=[
        pltpu.VMEM((EC,), _i32),
        pltpu.VMEM((NW, W), _i32),
        [pltpu.VMEM((EC,), _f32) for _ in range(4)],
        pltpu.VMEM((NS,), _f32),
        pltpu.VMEM((NS,), _f32),
        pltpu.VMEM((NS,), _f32),
        [pltpu.VMEM_SHARED((NPAD,), _f32) for _ in range(4)],
        [pltpu.VMEM_SHARED((NPAD,), _f32) for _ in range(4)],
        pltpu.SemaphoreType.DMA,
    ],
)
def _l1(src1d, dst2d, degp, xw1t, dinv_o, accp, srcflat, dstbuf, gbuf,
        nb0, nb1, zb, u_s, acc_s, sem):
    cid = lax.axis_index("c")
    sid = lax.axis_index("s")
    wid = cid * 16 + sid
    s0 = pl.ds(sid * NS, NS)
    pltpu.sync_copy(degp.at[0, s0], nb0)
    pltpu.sync_copy(degp.at[1, s0], nb1)
    for k in range(NS // 16):
        ds_ = pl.ds(k * 16, 16)
        nb0[ds_] = _rsqrt16(nb0[ds_] + nb1[ds_] + 1.0)

    @pl.when(cid == 0)
    def _():
        pltpu.sync_copy(nb0, dinv_o.at[s0])

    _fill(zb, NS, 0.0)
    for f in range(4):
        pltpu.sync_copy(xw1t.at[f, s0], nb1)
        for k in range(NS // 16):
            ds_ = pl.ds(k * 16, 16)
            nb1[ds_] = nb1[ds_] * nb0[ds_]
        pltpu.sync_copy(nb1, u_s[f].at[s0])
        pltpu.sync_copy(zb, acc_s[f].at[s0])
    pltpu.sync_copy(src1d.at[pl.ds(wid * EC, EC)], srcflat)
    pltpu.sync_copy(dst2d.at[pl.ds(wid * NW, NW)], dstbuf)
    plsc.subcore_barrier()
    _gather_scatter(u_s, acc_s, srcflat, gbuf, dstbuf, sem, 4)
    plsc.subcore_barrier()
    for f in range(4):
        pltpu.sync_copy(acc_s[f].at[s0], accp.at[cid, f, s0])


def _mid_layer(din, dout):
    @functools.partial(
        pl.kernel,
        mesh=_MESH,
        compiler_params=_SC_PARAMS,
        out_type=jax.ShapeDtypeStruct((2, dout, NPAD), _f32),
        scratch_types=[
            pltpu.VMEM((EC,), _i32),
            pltpu.VMEM((NW, W), _i32),
            [pltpu.VMEM((EC,), _f32) for _ in range(dout)],
            pltpu.VMEM((din, NS), _f32),
            pltpu.VMEM((NS,), _f32),
            pltpu.VMEM((NS,), _f32),
            pltpu.VMEM((NS,), _f32),
            pltpu.VMEM((din, dout, 16), _f32),
            pltpu.VMEM((din, 16), _f32),
            [pltpu.VMEM_SHARED((NPAD,), _f32) for _ in range(dout)],
            [pltpu.VMEM_SHARED((NPAD,), _f32) for _ in range(dout)],
            pltpu.SemaphoreType.DMA,
        ],
    )
    def _lay(src1d, dst2d, accp_in, dinv, wb, bb, accp_out, srcflat, dstbuf,
             gbuf, ybuf, t0, t1, dv, wbuf, bbuf, u_s, acc_s, sem):
        cid = lax.axis_index("c")
        sid = lax.axis_index("s")
        wid = cid * 16 + sid
        s0 = pl.ds(sid * NS, NS)
        pltpu.sync_copy(wb, wbuf)
        pltpu.sync_copy(bb, bbuf)
        pltpu.sync_copy(dinv.at[s0], dv)
        for f in range(din):
            pltpu.sync_copy(accp_in.at[0, f, s0], t0)
            pltpu.sync_copy(accp_in.at[1, f, s0], t1)
            for k in range(NS // 16):
                ds_ = pl.ds(k * 16, 16)
                ybuf[f, ds_] = jnp.maximum(
                    (t0[ds_] + t1[ds_]) * dv[ds_] + bbuf[f], 0.0)
        _fill(t1, NS, 0.0)
        for fp in range(dout):
            for k in range(NS // 16):
                ds_ = pl.ds(k * 16, 16)
                a = ybuf[0, ds_] * wbuf[0, fp]
                for f in range(1, din):
                    a = a + ybuf[f, ds_] * wbuf[f, fp]
                t0[ds_] = a * dv[ds_]
            pltpu.sync_copy(t0, u_s[fp].at[s0])
            pltpu.sync_copy(t1, acc_s[fp].at[s0])
        pltpu.sync_copy(src1d.at[pl.ds(wid * EC, EC)], srcflat)
        pltpu.sync_copy(dst2d.at[pl.ds(wid * NW, NW)], dstbuf)
        plsc.subcore_barrier()
        _gather_scatter(u_s, acc_s, srcflat, gbuf, dstbuf, sem, dout)
        plsc.subcore_barrier()
        for fp in range(dout):
            pltpu.sync_copy(acc_s[fp].at[s0], accp_out.at[cid, fp, s0])

    return _lay


_l2 = _mid_layer(4, 4)
_l3 = _mid_layer(4, 2)


@functools.partial(
    pl.kernel,
    mesh=_MESH,
    compiler_params=_SC_PARAMS,
    out_type=[
        jax.ShapeDtypeStruct((NPAD * 4,), _f32),
        jax.ShapeDtypeStruct((NPAD * 2,), _f32),
    ],
    scratch_types=[
        pltpu.VMEM((2, NG), _f32),
        pltpu.VMEM((NG,), _f32),
        pltpu.VMEM((NG,), _f32),
        pltpu.VMEM((NG,), _f32),
        pltpu.VMEM((2, 16), _f32),
        pltpu.VMEM((2, 4, 16), _f32),
        pltpu.VMEM((4, 16), _f32),
        pltpu.VMEM((NG * 4,), _f32),
        pltpu.VMEM((NG * 2,), _f32),
    ],
)
def _fin(accp3, dinv, b3b, wcb, bcb, out_o, y3_o,
         ybuf, t0, t1, dv, b3v, wcv, bcv, oil, yil):
    cid = lax.axis_index("c")
    sid = lax.axis_index("s")
    wid = cid * 16 + sid
    sg = pl.ds(wid * NG, NG)
    pltpu.sync_copy(b3b, b3v)
    pltpu.sync_copy(wcb, wcv)
    pltpu.sync_copy(bcb, bcv)
    pltpu.sync_copy(dinv.at[sg], dv)
    for f in range(2):
        pltpu.sync_copy(accp3.at[0, f, sg], t0)
        pltpu.sync_copy(accp3.at[1, f, sg], t1)
        for k in range(NG // 16):
            ds_ = pl.ds(k * 16, 16)
            ybuf[f, ds_] = jnp.maximum(
                (t0[ds_] + t1[ds_]) * dv[ds_] + b3v[f], 0.0)
    lane = lax.iota(_i32, 16)
    for k in range(NG // 16):
        ds_ = pl.ds(k * 16, 16)
        y0 = ybuf[0, ds_]
        y1 = ybuf[1, ds_]
        plsc.store_scatter(yil, [lane * 2 + (k * 32 + 0)], y0)
        plsc.store_scatter(yil, [lane * 2 + (k * 32 + 1)], y1)
        for c in range(4):
            oc = y0 * wcv[0, c] + y1 * wcv[1, c] + bcv[c]
            plsc.store_scatter(oil, [lane * 4 + (k * 64 + c)], oc)
    pltpu.sync_copy(oil, out_o.at[pl.ds(wid * NG * 4, NG * 4)])
    pltpu.sync_copy(yil, y3_o.at[pl.ds(wid * NG * 2, NG * 2)])


def kernel(x, edge_index, W1, b1, W2, b2, W3, b3, Wc, bc):
    xpad = jnp.zeros((NPAD, D_IN), _f32).at[:N].set(x)
    loops = jnp.arange(NPAD, dtype=_i32)
    npad_e = EPAD - E - NPAD
    padv = N + (jnp.arange(npad_e, dtype=_i32) % (NPAD - N))
    src = jnp.concatenate([edge_index[0], loops, padv])
    dst = jnp.concatenate([edge_index[1], loops, padv])
    dst2d = dst.reshape(EPAD // W, W)

    w2b = jnp.broadcast_to(W2[:, :, None], (4, 4, 16)).astype(_f32)
    w3b = jnp.broadcast_to(W3[:, :, None], (4, 2, 16)).astype(_f32)
    wcb = jnp.broadcast_to(Wc[:, :, None], (2, 4, 16)).astype(_f32)
    b1b = jnp.broadcast_to(b1[:, None], (4, 16)).astype(_f32)
    b2b = jnp.broadcast_to(b2[:, None], (4, 16)).astype(_f32)
    b3b = jnp.broadcast_to(b3[:, None], (2, 16)).astype(_f32)
    bcb = jnp.broadcast_to(bc[:, None], (4, 16)).astype(_f32)

    degp = _p1(dst2d)
    xw1t = _mm1(xpad, W1.T.astype(_f32))
    dinv, accp1 = _l1(src, dst2d, degp, xw1t)
    accp2 = _l2(src, dst2d, accp1, dinv, w2b, b1b)
    accp3 = _l3(src, dst2d, accp2, dinv, w3b, b2b)
    outf, y3f = _fin(accp3, dinv, b3b, wcb, bcb)
    out = outf.reshape(NPAD, 4)[:N]
    y3 = y3f.reshape(NPAD, 2)[:N]
    return (out, y3)


# speedup vs baseline: 56.7721x; 1.0011x over previous
import functools

import jax
import jax.numpy as jnp
from jax import lax
from jax.experimental import pallas as pl
from jax.experimental.pallas import tpu as pltpu
from jax.experimental.pallas import tpu_sc as plsc

N = 10000
D_IN = 128
E = 320000
NPAD = 10240
W = 128
EPAD = 331776
NW = EPAD // (32 * W)
EC = NW * W
NS = NPAD // 16
NG = NPAD // 32

_MESH = plsc.VectorSubcoreMesh(
    core_axis_name="c", subcore_axis_name="s", num_cores=2, num_subcores=16)
_SC_PARAMS = pltpu.CompilerParams(
    use_tc_tiling_on_sc=False, needs_layout_passes=False)

_f32 = jnp.float32
_i32 = jnp.int32


def _rsqrt16(d):
    i = lax.bitcast_convert_type(d, _i32)
    i = jnp.int32(0x5F3759DF) - lax.shift_right_arithmetic(i, 1)
    y = lax.bitcast_convert_type(i, _f32)
    h = d * 0.5
    for _ in range(3):
        y = y * (1.5 - h * y * y)
    return y


def _fill(ref, n, val):
    v = jnp.full((16,), val, _f32)
    for k in range(n // 16):
        ref[pl.ds(k * 16, 16)] = v


def _gather_scatter(u_s, acc_s, srcflat, gbuf, dstbuf, sem, nf):
    gd = [pltpu.async_copy(u_s[f].at[srcflat], gbuf[f], sem) for f in range(nf)]
    for f in range(nf):
        gd[f].wait()
        sd = [
            pltpu.async_copy(
                gbuf[f].at[pl.ds(w * W, W)],
                acc_s[f].at[dstbuf.at[w]],
                sem, add=True)
            for w in range(NW)
        ]
        for d in sd:
            d.wait()


_MMBLK = 1024


def _mm1_body(x_ref, w1t_ref, o_ref):
    xb = x_ref[...]
    for f in range(4):
        wrow = w1t_ref[f, :]
        o_ref[pl.ds(f, 1), :] = jnp.sum(xb * wrow[None, :], axis=1)[None, :]


_mm1 = pl.pallas_call(
    _mm1_body,
    grid=(NPAD // _MMBLK,),
    in_specs=[
        pl.BlockSpec((_MMBLK, D_IN), lambda i: (i, 0)),
        pl.BlockSpec((4, D_IN), lambda i: (0, 0)),
    ],
    out_specs=pl.BlockSpec((4, _MMBLK), lambda i: (0, i)),
    out_shape=jax.ShapeDtypeStruct((4, NPAD), _f32),
)


@functools.partial(
    pl.kernel,
    mesh=_MESH,
    compiler_params=_SC_PARAMS,
    out_type=jax.ShapeDtypeStruct((2, NPAD), _f32),
    scratch_types=[
        pltpu.VMEM((NW, W), _i32),
        pltpu.VMEM((W,), _f32),
        pltpu.VMEM((NS,), _f32),
        pltpu.VMEM_SHARED((NPAD,), _f32),
        pltpu.SemaphoreType.DMA,
    ],
)
def _p1(dst2d, degp, dstbuf, ones, zb, degs, sem):
    cid = lax.axis_index("c")
    sid = lax.axis_index("s")
    wid = cid * 16 + sid
    s0 = pl.ds(sid * NS, NS)
    _fill(ones, W, 1.0)
    _fill(zb, NS, 0.0)
    pltpu.sync_copy(zb, degs.at[s0])
    pltpu.sync_copy(dst2d.at[pl.ds(wid * NW, NW)], dstbuf)
    plsc.subcore_barrier()
    sd = [
        pltpu.async_copy(ones, degs.at[dstbuf.at[w]], sem, add=True)
        for w in range(NW)
    ]
    for d in sd:
        d.wait()
    plsc.subcore_barrier()
    pltpu.sync_copy(degs.at[s0], degp.at[cid, s0])


@functools.partial(
    pl.kernel,
    mesh=_MESH,
    compiler_params=_SC_PARAMS,
    out_type=[
        jax.ShapeDtypeStruct((NPAD,), _f32),
        jax.ShapeDtypeStruct((2, 4, NPAD), _f32),
    ],
    scratch_types=[
        pltpu.VMEM((EC,), _i32),
        pltpu.VMEM((NW, W), _i32),
        [pltpu.VMEM((EC,), _f32) for _ in range(4)],
        pltpu.VMEM((NS,), _f32),
        pltpu.VMEM((NS,), _f32),
        pltpu.VMEM((NS,), _f32),
        [pltpu.VMEM_SHARED((NPAD,), _f32) for _ in range(4)],
        [pltpu.VMEM_SHARED((NPAD,), _f32) for _ in range(4)],
        pltpu.SemaphoreType.DMA,
    ],
)
def _l1(src1d, dst2d, degp, xw1t, dinv_o, accp, srcflat, dstbuf, gbuf,
        nb0, nb1, zb, u_s, acc_s, sem):
    cid = lax.axis_index("c")
    sid = lax.axis_index("s")
    wid = cid * 16 + sid
    s0 = pl.ds(sid * NS, NS)
    pltpu.sync_copy(degp.at[0, s0], nb0)
    pltpu.sync_copy(degp.at[1, s0], nb1)
    for k in range(NS // 16):
        ds_ = pl.ds(k * 16, 16)
        nb0[ds_] = _rsqrt16(nb0[ds_] + nb1[ds_])

    @pl.when(cid == 0)
    def _():
        pltpu.sync_copy(nb0, dinv_o.at[s0])

    _fill(zb, NS, 0.0)
    for f in range(4):
        pltpu.sync_copy(xw1t.at[f, s0], nb1)
        for k in range(NS // 16):
            ds_ = pl.ds(k * 16, 16)
            nb1[ds_] = nb1[ds_] * nb0[ds_]
        pltpu.sync_copy(nb1, u_s[f].at[s0])
        pltpu.sync_copy(zb, acc_s[f].at[s0])
    pltpu.sync_copy(src1d.at[pl.ds(wid * EC, EC)], srcflat)
    pltpu.sync_copy(dst2d.at[pl.ds(wid * NW, NW)], dstbuf)
    plsc.subcore_barrier()
    _gather_scatter(u_s, acc_s, srcflat, gbuf, dstbuf, sem, 4)
    plsc.subcore_barrier()
    for f in range(4):
        pltpu.sync_copy(acc_s[f].at[s0], accp.at[cid, f, s0])


def _mid_layer(din, dout):
    @functools.partial(
        pl.kernel,
        mesh=_MESH,
        compiler_params=_SC_PARAMS,
        out_type=jax.ShapeDtypeStruct((2, dout, NPAD), _f32),
        scratch_types=[
            pltpu.VMEM((EC,), _i32),
            pltpu.VMEM((NW, W), _i32),
            [pltpu.VMEM((EC,), _f32) for _ in range(dout)],
            pltpu.VMEM((din, NS), _f32),
            pltpu.VMEM((NS,), _f32),
            pltpu.VMEM((NS,), _f32),
            pltpu.VMEM((NS,), _f32),
            pltpu.VMEM((din, dout, 16), _f32),
            pltpu.VMEM((din, 16), _f32),
            [pltpu.VMEM_SHARED((NPAD,), _f32) for _ in range(dout)],
            [pltpu.VMEM_SHARED((NPAD,), _f32) for _ in range(dout)],
            pltpu.SemaphoreType.DMA,
        ],
    )
    def _lay(src1d, dst2d, accp_in, dinv, wb, bb, accp_out, srcflat, dstbuf,
             gbuf, ybuf, t0, t1, dv, wbuf, bbuf, u_s, acc_s, sem):
        cid = lax.axis_index("c")
        sid = lax.axis_index("s")
        wid = cid * 16 + sid
        s0 = pl.ds(sid * NS, NS)
        pltpu.sync_copy(wb, wbuf)
        pltpu.sync_copy(bb, bbuf)
        pltpu.sync_copy(dinv.at[s0], dv)
        for f in range(din):
            pltpu.sync_copy(accp_in.at[0, f, s0], t0)
            pltpu.sync_copy(accp_in.at[1, f, s0], t1)
            for k in range(NS // 16):
                ds_ = pl.ds(k * 16, 16)
                ybuf[f, ds_] = jnp.maximum(
                    (t0[ds_] + t1[ds_]) * dv[ds_] + bbuf[f], 0.0)
        _fill(t1, NS, 0.0)
        for fp in range(dout):
            for k in range(NS // 16):
                ds_ = pl.ds(k * 16, 16)
                a = ybuf[0, ds_] * wbuf[0, fp]
                for f in range(1, din):
                    a = a + ybuf[f, ds_] * wbuf[f, fp]
                t0[ds_] = a * dv[ds_]
            pltpu.sync_copy(t0, u_s[fp].at[s0])
            pltpu.sync_copy(t1, acc_s[fp].at[s0])
        pltpu.sync_copy(src1d.at[pl.ds(wid * EC, EC)], srcflat)
        pltpu.sync_copy(dst2d.at[pl.ds(wid * NW, NW)], dstbuf)
        plsc.subcore_barrier()
        _gather_scatter(u_s, acc_s, srcflat, gbuf, dstbuf, sem, dout)
        plsc.subcore_barrier()
        for fp in range(dout):
            pltpu.sync_copy(acc_s[fp].at[s0], accp_out.at[cid, fp, s0])

    return _lay


_l2 = _mid_layer(4, 4)
_l3 = _mid_layer(4, 2)


@functools.partial(
    pl.kernel,
    mesh=_MESH,
    compiler_params=_SC_PARAMS,
    out_type=[
        jax.ShapeDtypeStruct((NPAD * 4,), _f32),
        jax.ShapeDtypeStruct((NPAD * 2,), _f32),
    ],
    scratch_types=[
        pltpu.VMEM((2, NG), _f32),
        pltpu.VMEM((NG,), _f32),
        pltpu.VMEM((NG,), _f32),
        pltpu.VMEM((NG,), _f32),
        pltpu.VMEM((2, 16), _f32),
        pltpu.VMEM((2, 4, 16), _f32),
        pltpu.VMEM((4, 16), _f32),
        pltpu.VMEM((NG * 4,), _f32),
        pltpu.VMEM((NG * 2,), _f32),
    ],
)
def _fin(accp3, dinv, b3b, wcb, bcb, out_o, y3_o,
         ybuf, t0, t1, dv, b3v, wcv, bcv, oil, yil):
    cid = lax.axis_index("c")
    sid = lax.axis_index("s")
    wid = cid * 16 + sid
    sg = pl.ds(wid * NG, NG)
    pltpu.sync_copy(b3b, b3v)
    pltpu.sync_copy(wcb, wcv)
    pltpu.sync_copy(bcb, bcv)
    pltpu.sync_copy(dinv.at[sg], dv)
    for f in range(2):
        pltpu.sync_copy(accp3.at[0, f, sg], t0)
        pltpu.sync_copy(accp3.at[1, f, sg], t1)
        for k in range(NG // 16):
            ds_ = pl.ds(k * 16, 16)
            ybuf[f, ds_] = jnp.maximum(
                (t0[ds_] + t1[ds_]) * dv[ds_] + b3v[f], 0.0)
    lane = lax.iota(_i32, 16)
    for k in range(NG // 16):
        ds_ = pl.ds(k * 16, 16)
        y0 = ybuf[0, ds_]
        y1 = ybuf[1, ds_]
        plsc.store_scatter(yil, [lane * 2 + (k * 32 + 0)], y0)
        plsc.store_scatter(yil, [lane * 2 + (k * 32 + 1)], y1)
        for c in range(4):
            oc = y0 * wcv[0, c] + y1 * wcv[1, c] + bcv[c]
            plsc.store_scatter(oil, [lane * 4 + (k * 64 + c)], oc)
    pltpu.sync_copy(oil, out_o.at[pl.ds(wid * NG * 4, NG * 4)])
    pltpu.sync_copy(yil, y3_o.at[pl.ds(wid * NG * 2, NG * 2)])


def kernel(x, edge_index, W1, b1, W2, b2, W3, b3, Wc, bc):
    xpad = jnp.zeros((NPAD, D_IN), _f32).at[:N].set(x)
    loops = jnp.arange(NPAD, dtype=_i32)
    npad_e = EPAD - E - NPAD
    padv = N + (jnp.arange(npad_e, dtype=_i32) % (NPAD - N))
    src = jnp.concatenate([edge_index[0], loops, padv])
    dst = jnp.concatenate([edge_index[1], loops, padv])
    dst2d = dst.reshape(EPAD // W, W)

    w2b = jnp.broadcast_to(W2[:, :, None], (4, 4, 16)).astype(_f32)
    w3b = jnp.broadcast_to(W3[:, :, None], (4, 2, 16)).astype(_f32)
    wcb = jnp.broadcast_to(Wc[:, :, None], (2, 4, 16)).astype(_f32)
    b1b = jnp.broadcast_to(b1[:, None], (4, 16)).astype(_f32)
    b2b = jnp.broadcast_to(b2[:, None], (4, 16)).astype(_f32)
    b3b = jnp.broadcast_to(b3[:, None], (2, 16)).astype(_f32)
    bcb = jnp.broadcast_to(bc[:, None], (4, 16)).astype(_f32)

    degp = _p1(dst2d)
    xw1t = _mm1(xpad, W1.T.astype(_f32))
    dinv, accp1 = _l1(src, dst2d, degp, xw1t)
    accp2 = _l2(src, dst2d, accp1, dinv, w2b, b1b)
    accp3 = _l3(src, dst2d, accp2, dinv, w3b, b2b)
    outf, y3f = _fin(accp3, dinv, b3b, wcb, bcb)
    out = outf.reshape(NPAD, 4)[:N]
    y3 = y3f.reshape(NPAD, 2)[:N]
    return (out, y3)
